# baseline (speedup 1.0000x reference)
import functools

import jax
import jax.numpy as jnp
from jax import lax
from jax.experimental import pallas as pl
from jax.experimental.pallas import tpu as pltpu
from jax.experimental.pallas import tpu_sc as plsc

EMB = 64
NC = 2
NS = 16
NW = NC * NS
NBUF = 8
D = 4


def _gather_body(t_hbm, table_hbm, out_hbm, idx_v, rows_v, gsem, ssem):
    wid = lax.axis_index("s") * NC + lax.axis_index("c")
    n_i, seq = idx_v.shape
    base = wid * n_i
    pltpu.sync_copy(t_hbm.at[pl.ds(base, n_i)], idx_v)

    def gather(li, slot):
        return pltpu.make_async_copy(
            table_hbm.at[idx_v.at[li]], rows_v.at[slot], gsem.at[slot])

    def store(li, slot):
        return pltpu.make_async_copy(
            rows_v.at[slot], out_hbm.at[base + li], ssem.at[slot])

    for b in range(D):
        gather(b, b).start()

    def body(li, carry):
        slot = lax.rem(li, NBUF)
        gather(li, slot).wait()
        store(li, slot).start()
        nxt = li + D
        nslot = lax.rem(nxt, NBUF)

        @pl.when(nxt < n_i)
        def _():
            @pl.when(nxt >= NBUF)
            def _():
                store(nxt - NBUF, nslot).wait()

            gather(nxt, nslot).start()

        return carry

    lax.fori_loop(0, n_i, body, 0, unroll=False)

    for b in range(NBUF):
        li = n_i - NBUF + b
        store(li, li % NBUF).wait()


def kernel(t, pos_embeddings):
    B, S = t.shape
    V, E = pos_embeddings.shape
    assert E == EMB and B % NW == 0
    n_i = B // NW

    mesh = plsc.VectorSubcoreMesh(core_axis_name="c", subcore_axis_name="s")

    run = functools.partial(
        pl.kernel,
        out_type=jax.ShapeDtypeStruct((B, S, EMB), jnp.float32),
        mesh=mesh,
        scratch_types=[
            pltpu.VMEM((n_i, S), jnp.int32),
            pltpu.VMEM((NBUF, S, EMB), jnp.float32),
            pltpu.SemaphoreType.DMA((NBUF,)),
            pltpu.SemaphoreType.DMA((NBUF,)),
        ],
        compiler_params=pltpu.CompilerParams(use_tc_tiling_on_sc=False),
    )(_gather_body)

    return run(t, pos_embeddings)


# speedup vs baseline: 1.7746x; 1.7746x over previous
import functools

import jax
import jax.numpy as jnp
from jax import lax
from jax.experimental import pallas as pl
from jax.experimental.pallas import tpu as pltpu
from jax.experimental.pallas import tpu_sc as plsc

EMB = 64
NC = 2
NS = 16
NW = NC * NS
NBUF = 8
D = 4


def _gather_body(t_hbm, table_hbm, out_hbm, idx_v, rows_v, gsem, ssem):
    wid = lax.axis_index("s") * NC + lax.axis_index("c")
    n_i, seq = idx_v.shape
    base = wid * n_i
    pltpu.sync_copy(t_hbm.at[pl.ds(base, n_i)], idx_v)

    def gather(li, slot):
        return pltpu.make_async_copy(
            table_hbm.at[idx_v.at[li]], rows_v.at[slot], gsem.at[slot])

    def store(li, slot):
        return pltpu.make_async_copy(
            rows_v.at[slot],
            out_hbm.at[base + li, pl.ds(0, seq), pl.ds(0, rows_v.shape[2])],
            ssem.at[slot])

    for b in range(D):
        gather(b, b).start()

    def body(li, carry):
        slot = lax.rem(li, NBUF)
        gather(li, slot).wait()
        store(li, slot).start()
        nxt = li + D
        nslot = lax.rem(nxt, NBUF)

        @pl.when(nxt < n_i)
        def _():
            @pl.when(nxt >= NBUF)
            def _():
                store(nxt - NBUF, nslot).wait()

            gather(nxt, nslot).start()

        return carry

    lax.fori_loop(0, n_i, body, 0, unroll=False)

    for b in range(NBUF):
        li = n_i - NBUF + b
        store(li, li % NBUF).wait()


def kernel(t, pos_embeddings):
    B, S = t.shape
    V, E = pos_embeddings.shape
    assert E == EMB and B % NW == 0
    n_i = B // NW

    mesh = plsc.VectorSubcoreMesh(core_axis_name="c", subcore_axis_name="s")

    s_pad = (S + 7) // 8 * 8
    run = functools.partial(
        pl.kernel,
        out_type=jax.ShapeDtypeStruct((B, s_pad, 2 * EMB), jnp.float32),
        mesh=mesh,
        scratch_types=[
            pltpu.VMEM((n_i, S), jnp.int32),
            pltpu.VMEM((NBUF, S, EMB), jnp.float32),
            pltpu.SemaphoreType.DMA((NBUF,)),
            pltpu.SemaphoreType.DMA((NBUF,)),
        ],
        compiler_params=pltpu.CompilerParams(use_tc_tiling_on_sc=False),
    )(_gather_body)

    out = run(t, pos_embeddings)
    return out[:, :S, :EMB]


# speedup vs baseline: 1.8133x; 1.0218x over previous
import functools

import jax
import jax.numpy as jnp
from jax import lax
from jax.experimental import pallas as pl
from jax.experimental.pallas import tpu as pltpu
from jax.experimental.pallas import tpu_sc as plsc

EMB = 64
NC = 2
NS = 16
NW = NC * NS
NBUF = 8
D = 6


def _gather_body(t_hbm, table_hbm, out_hbm, idx_v, rows_v, gsem, ssem):
    wid = lax.axis_index("s") * NC + lax.axis_index("c")
    n_i, seq = idx_v.shape
    base = wid * n_i
    pltpu.sync_copy(t_hbm.at[pl.ds(base, n_i)], idx_v)

    def gather(li, slot):
        return pltpu.make_async_copy(
            table_hbm.at[idx_v.at[li]], rows_v.at[slot], gsem.at[slot])

    def store(li, slot):
        return pltpu.make_async_copy(
            rows_v.at[slot],
            out_hbm.at[base + li, pl.ds(0, seq), pl.ds(0, rows_v.shape[2])],
            ssem.at[slot])

    for b in range(D):
        gather(b, b).start()

    def body(li, carry):
        slot = lax.rem(li, NBUF)
        gather(li, slot).wait()
        store(li, slot).start()
        nxt = li + D
        nslot = lax.rem(nxt, NBUF)

        @pl.when(nxt < n_i)
        def _():
            @pl.when(nxt >= NBUF)
            def _():
                store(nxt - NBUF, nslot).wait()

            gather(nxt, nslot).start()

        return carry

    lax.fori_loop(0, n_i, body, 0, unroll=False)

    for b in range(NBUF):
        li = n_i - NBUF + b
        store(li, li % NBUF).wait()


def kernel(t, pos_embeddings):
    B, S = t.shape
    V, E = pos_embeddings.shape
    assert E == EMB and B % NW == 0
    n_i = B // NW

    mesh = plsc.VectorSubcoreMesh(core_axis_name="c", subcore_axis_name="s")

    s_pad = (S + 7) // 8 * 8
    run = functools.partial(
        pl.kernel,
        out_type=jax.ShapeDtypeStruct((B, s_pad, 2 * EMB), jnp.float32),
        mesh=mesh,
        scratch_types=[
            pltpu.VMEM((n_i, S), jnp.int32),
            pltpu.VMEM((NBUF, S, EMB), jnp.float32),
            pltpu.SemaphoreType.DMA((NBUF,)),
            pltpu.SemaphoreType.DMA((NBUF,)),
        ],
        compiler_params=pltpu.CompilerParams(use_tc_tiling_on_sc=False),
    )(_gather_body)

    out = run(t, pos_embeddings)
    return out[:, :S, :EMB]
